# baseline (speedup 1.0000x reference)
import functools

import jax
import jax.numpy as jnp
import numpy as np
from jax import lax
from jax.experimental import pallas as pl
from jax.experimental.pallas import tpu as pltpu
from jax.experimental.pallas import tpu_sc as plsc

_EPS = float(np.finfo(np.float32).tiny)
_K = 10
_TAU = 0.1
_ROWS = 64
_COLS = 4096
_L = 16
_NC = 2
_NS = 16
_NW = _NC * _NS
_RPW = _ROWS // _NW
_CHUNKS = _COLS // _L
_WSIZE = _RPW * _COLS


def _lane_perm(v, d):
    idx = lax.iota(jnp.int32, _L) ^ d
    return v.at[idx].get(mode="promise_in_bounds")


def _allmax(v):
    for d in (8, 4, 2, 1):
        v = jnp.maximum(v, _lane_perm(v, d))
    return v


def _allsum(v):
    for d in (8, 4, 2, 1):
        v = v + _lane_perm(v, d)
    return v


def _sc_body(w_hbm, z_hbm, out_hbm, wbuf, zbuf, khbuf):
    wid = lax.axis_index("s") * _NC + lax.axis_index("c")
    base = wid * _WSIZE
    pltpu.sync_copy(w_hbm.at[pl.ds(base, _WSIZE)], wbuf)
    pltpu.sync_copy(z_hbm.at[pl.ds(base, _WSIZE)], zbuf)

    for r in range(_RPW):
        roff = r * _COLS

        def p1(j, mx):
            off = roff + j * _L
            a = (wbuf[pl.ds(off, _L)] + zbuf[pl.ds(off, _L)]) / _TAU
            wbuf[pl.ds(off, _L)] = a
            return jnp.maximum(mx, a)

        mx = lax.fori_loop(0, _CHUNKS, p1,
                           jnp.full((_L,), -jnp.inf, jnp.float32))
        mv = _allmax(mx)

        def p2(j, sm):
            off = roff + j * _L
            b = jnp.exp(wbuf[pl.ds(off, _L)] - mv)
            wbuf[pl.ds(off, _L)] = b
            khbuf[pl.ds(off, _L)] = jnp.zeros((_L,), jnp.float32)
            return sm + b

        sv = lax.fori_loop(0, _CHUNKS, p2, jnp.zeros((_L,), jnp.float32))
        ssum_v = _allsum(sv)
        rmax_v = jnp.ones((_L,), jnp.float32)

        for _ in range(_K - 1):
            inv_s = 1.0 / ssum_v
            inv_r = 1.0 / rmax_v

            def it(j, carry):
                nmx, nsm = carry
                off = roff + j * _L
                b = wbuf[pl.ds(off, _L)]
                sj = b * inv_s
                plsc.addupdate(khbuf.at[pl.ds(off, _L)], sj)
                m = jnp.maximum(1.0 - sj, _EPS)
                m2 = m * m
                m4 = m2 * m2
                bn = (b * inv_r) * (m4 * m4 * m2)
                wbuf[pl.ds(off, _L)] = bn
                return jnp.maximum(nmx, bn), nsm + bn

            nmx, nsm = lax.fori_loop(
                0, _CHUNKS, it,
                (jnp.zeros((_L,), jnp.float32), jnp.zeros((_L,), jnp.float32)))
            ssum_v = _allsum(nsm)
            rmax_v = _allmax(nmx)

        inv_s = 1.0 / ssum_v

        def it_last(j, acc):
            off = roff + j * _L
            plsc.addupdate(khbuf.at[pl.ds(off, _L)],
                           wbuf[pl.ds(off, _L)] * inv_s)
            return acc

        lax.fori_loop(0, _CHUNKS, it_last, jnp.int32(0))

    pltpu.sync_copy(khbuf, out_hbm.at[pl.ds(base, _WSIZE)])


_run = pl.kernel(
    _sc_body,
    out_type=jax.ShapeDtypeStruct((_ROWS * _COLS,), jnp.float32),
    mesh=plsc.VectorSubcoreMesh(core_axis_name="c", subcore_axis_name="s"),
    scratch_types=[
        pltpu.VMEM((_WSIZE,), jnp.float32),
        pltpu.VMEM((_WSIZE,), jnp.float32),
        pltpu.VMEM((_WSIZE,), jnp.float32),
    ],
)

def _rotl32(x, d):
    return ((x << np.uint32(d)) | (x >> np.uint32(32 - d))).astype(np.uint32)


def _threefry2x32(k0, k1, x0, x1):
    x0 = x0.astype(np.uint32).copy()
    x1 = x1.astype(np.uint32).copy()
    ks = [np.uint32(k0), np.uint32(k1),
          np.uint32(np.uint32(k0) ^ np.uint32(k1) ^ np.uint32(0x1BD11BDA))]
    rots = [[13, 15, 26, 6], [17, 29, 16, 24]]
    with np.errstate(over="ignore"):
        x0 += ks[0]
        x1 += ks[1]
        for i in range(5):
            for r in rots[i % 2]:
                x0 += x1
                x1 = _rotl32(x1, r)
                x1 ^= x0
            x0 += ks[(i + 1) % 3]
            x1 += ks[(i + 2) % 3] + np.uint32(i + 1)
    return x0, x1


_Z_CONST = None


def _gumbel_z():
    global _Z_CONST
    if _Z_CONST is None:
        k0, k1 = _threefry2x32(0, 0, np.zeros(1, np.uint32),
                               np.full(1, 12345, np.uint32))
        counts = np.arange(_ROWS * _COLS, dtype=np.uint64)
        hi = (counts >> np.uint64(32)).astype(np.uint32)
        lo = (counts & np.uint64(0xFFFFFFFF)).astype(np.uint32)
        o0, o1 = _threefry2x32(k0[0], k1[0], hi, lo)
        bits = o0 ^ o1
        f = ((bits >> np.uint32(9)) | np.uint32(0x3F800000)).view(np.float32)
        u = (f - np.float32(1.0)) * np.float32(1.0 - _EPS) + np.float32(_EPS)
        u = np.maximum(np.float32(_EPS), u)
        _Z_CONST = np.log(-np.log(u))
    return _Z_CONST


def kernel(logits):
    w = jnp.reshape(logits, (_ROWS * _COLS,))
    out = _run(w, jnp.asarray(_gumbel_z()))
    return jnp.reshape(out, (_ROWS, _COLS, 1))


# speedup vs baseline: 1.0067x; 1.0067x over previous
import functools

import jax
import jax.numpy as jnp
import numpy as np
from jax import lax
from jax.experimental import pallas as pl
from jax.experimental.pallas import tpu as pltpu
from jax.experimental.pallas import tpu_sc as plsc

_EPS = float(np.finfo(np.float32).tiny)
_K = 10
_TAU = 0.1
_ROWS = 64
_COLS = 4096
_L = 16
_NC = 2
_NS = 16
_NW = _NC * _NS
_RPW = _ROWS // _NW
_CHUNKS = _COLS // _L
_WSIZE = _RPW * _COLS
_UNROLL = 8


def _lane_perm(v, d):
    idx = lax.iota(jnp.int32, _L) ^ d
    return v.at[idx].get(mode="promise_in_bounds")


def _allmax(v):
    for d in (8, 4, 2, 1):
        v = jnp.maximum(v, _lane_perm(v, d))
    return v


def _allsum(v):
    for d in (8, 4, 2, 1):
        v = v + _lane_perm(v, d)
    return v


def _sc_body(w_hbm, z_hbm, out_hbm, wbuf, zbuf, khbuf):
    wid = lax.axis_index("s") * _NC + lax.axis_index("c")
    base = wid * _WSIZE
    pltpu.sync_copy(w_hbm.at[pl.ds(base, _WSIZE)], wbuf)
    pltpu.sync_copy(z_hbm.at[pl.ds(base, _WSIZE)], zbuf)

    for r in range(_RPW):
        roff = r * _COLS

        @plsc.parallel_loop(roff, roff + _COLS, _L, unroll=_UNROLL,
                            carry=jnp.full((_L,), -jnp.inf, jnp.float32))
        def mx(off, mx):
            a = (wbuf[pl.ds(off, _L)] + zbuf[pl.ds(off, _L)]) / _TAU
            wbuf[pl.ds(off, _L)] = a
            return jnp.maximum(mx, a)

        mv = _allmax(mx)

        @plsc.parallel_loop(roff, roff + _COLS, _L, unroll=_UNROLL,
                            carry=jnp.zeros((_L,), jnp.float32))
        def sv(off, sm):
            b = jnp.exp(wbuf[pl.ds(off, _L)] - mv)
            wbuf[pl.ds(off, _L)] = b
            khbuf[pl.ds(off, _L)] = jnp.zeros((_L,), jnp.float32)
            return sm + b

        ssum_v = _allsum(sv)
        rmax_v = jnp.ones((_L,), jnp.float32)

        for _ in range(_K - 1):
            inv_s = 1.0 / ssum_v
            inv_r = 1.0 / rmax_v

            @plsc.parallel_loop(
                roff, roff + _COLS, _L, unroll=_UNROLL,
                carry=(jnp.zeros((_L,), jnp.float32),
                       jnp.zeros((_L,), jnp.float32)))
            def acc(off, carry):
                nmx, nsm = carry
                b = wbuf[pl.ds(off, _L)]
                sj = b * inv_s
                plsc.addupdate(khbuf.at[pl.ds(off, _L)], sj)
                m = jnp.maximum(1.0 - sj, _EPS)
                m2 = m * m
                m4 = m2 * m2
                bn = (b * inv_r) * (m4 * m4 * m2)
                wbuf[pl.ds(off, _L)] = bn
                return jnp.maximum(nmx, bn), nsm + bn

            nmx, nsm = acc
            ssum_v = _allsum(nsm)
            rmax_v = _allmax(nmx)

        inv_s = 1.0 / ssum_v

        @plsc.parallel_loop(roff, roff + _COLS, _L, unroll=_UNROLL)
        def _(off):
            plsc.addupdate(khbuf.at[pl.ds(off, _L)],
                           wbuf[pl.ds(off, _L)] * inv_s)

    pltpu.sync_copy(khbuf, out_hbm.at[pl.ds(base, _WSIZE)])


_run = pl.kernel(
    _sc_body,
    out_type=jax.ShapeDtypeStruct((_ROWS * _COLS,), jnp.float32),
    mesh=plsc.VectorSubcoreMesh(core_axis_name="c", subcore_axis_name="s"),
    scratch_types=[
        pltpu.VMEM((_WSIZE,), jnp.float32),
        pltpu.VMEM((_WSIZE,), jnp.float32),
        pltpu.VMEM((_WSIZE,), jnp.float32),
    ],
)

def _rotl32(x, d):
    return ((x << np.uint32(d)) | (x >> np.uint32(32 - d))).astype(np.uint32)


def _threefry2x32(k0, k1, x0, x1):
    x0 = x0.astype(np.uint32).copy()
    x1 = x1.astype(np.uint32).copy()
    ks = [np.uint32(k0), np.uint32(k1),
          np.uint32(np.uint32(k0) ^ np.uint32(k1) ^ np.uint32(0x1BD11BDA))]
    rots = [[13, 15, 26, 6], [17, 29, 16, 24]]
    with np.errstate(over="ignore"):
        x0 += ks[0]
        x1 += ks[1]
        for i in range(5):
            for r in rots[i % 2]:
                x0 += x1
                x1 = _rotl32(x1, r)
                x1 ^= x0
            x0 += ks[(i + 1) % 3]
            x1 += ks[(i + 2) % 3] + np.uint32(i + 1)
    return x0, x1


_Z_CONST = None


def _gumbel_z():
    global _Z_CONST
    if _Z_CONST is None:
        k0, k1 = _threefry2x32(0, 0, np.zeros(1, np.uint32),
                               np.full(1, 12345, np.uint32))
        counts = np.arange(_ROWS * _COLS, dtype=np.uint64)
        hi = (counts >> np.uint64(32)).astype(np.uint32)
        lo = (counts & np.uint64(0xFFFFFFFF)).astype(np.uint32)
        o0, o1 = _threefry2x32(k0[0], k1[0], hi, lo)
        bits = o0 ^ o1
        f = ((bits >> np.uint32(9)) | np.uint32(0x3F800000)).view(np.float32)
        u = (f - np.float32(1.0)) * np.float32(1.0 - _EPS) + np.float32(_EPS)
        u = np.maximum(np.float32(_EPS), u)
        _Z_CONST = np.log(-np.log(u))
    return _Z_CONST


def kernel(logits):
    w = jnp.reshape(logits, (_ROWS * _COLS,))
    out = _run(w, jnp.asarray(_gumbel_z()))
    return jnp.reshape(out, (_ROWS, _COLS, 1))


# speedup vs baseline: 2.6213x; 2.6039x over previous
import functools

import jax
import jax.numpy as jnp
import numpy as np
from jax import lax
from jax.experimental import pallas as pl
from jax.experimental.pallas import tpu as pltpu
from jax.experimental.pallas import tpu_sc as plsc

_EPS = float(np.finfo(np.float32).tiny)
_K = 10
_TAU = 0.1
_ROWS = 64
_COLS = 4096
_L = 16
_NC = 2
_NS = 16
_NW = _NC * _NS
_RPW = _ROWS // _NW
_CHUNKS = _COLS // _L
_WSIZE = _RPW * _COLS
_MU = 8


def _lane_perm(v, d):
    idx = lax.iota(jnp.int32, _L) ^ d
    return v.at[idx].get(mode="promise_in_bounds")


def _allmax(v):
    for d in (8, 4, 2, 1):
        v = jnp.maximum(v, _lane_perm(v, d))
    return v


def _allsum(v):
    for d in (8, 4, 2, 1):
        v = v + _lane_perm(v, d)
    return v


def _sc_body(a_hbm, out_hbm, wbuf, khbuf):
    wid = lax.axis_index("s") * _NC + lax.axis_index("c")
    base = wid * _WSIZE
    pltpu.sync_copy(a_hbm.at[pl.ds(base, _WSIZE)], wbuf)

    group = _L * _MU

    @plsc.parallel_loop(0, _COLS, group,
                        carry=(jnp.full((_L,), -jnp.inf, jnp.float32),
                               jnp.full((_L,), -jnp.inf, jnp.float32)))
    def mx(off, carry):
        out = []
        for r in range(_RPW):
            a = [wbuf[pl.ds(r * _COLS + off + u * _L, _L)]
                 for u in range(_MU)]
            t = a
            while len(t) > 1:
                t = [jnp.maximum(t[i], t[i + 1]) for i in range(0, len(t), 2)]
            out.append(jnp.maximum(carry[r], t[0]))
        return tuple(out)

    mv = [_allmax(m) for m in mx]

    @plsc.parallel_loop(0, _COLS, group,
                        carry=(jnp.zeros((_L,), jnp.float32),
                               jnp.zeros((_L,), jnp.float32)))
    def sv(off, carry):
        out = []
        for r in range(_RPW):
            b = [jnp.exp(wbuf[pl.ds(r * _COLS + off + u * _L, _L)] - mv[r])
                 for u in range(_MU)]
            for u in range(_MU):
                wbuf[pl.ds(r * _COLS + off + u * _L, _L)] = b[u]
            t = b
            while len(t) > 1:
                t = [t[i] + t[i + 1] for i in range(0, len(t), 2)]
            out.append(carry[r] + t[0])
        return tuple(out)

    ssum_v = [_allsum(s) for s in sv]

    for it in range(_K - 1):
        inv_s = [1.0 / s for s in ssum_v]

        @plsc.parallel_loop(0, _COLS, group,
                            carry=(jnp.zeros((_L,), jnp.float32),
                                   jnp.zeros((_L,), jnp.float32)))
        def nsm(off, carry):
            out = []
            for r in range(_RPW):
                bs = [wbuf[pl.ds(r * _COLS + off + u * _L, _L)]
                      for u in range(_MU)]
                bn = []
                for u in range(_MU):
                    b = bs[u]
                    sj = b * inv_s[r]
                    if it == 0:
                        khbuf[pl.ds(r * _COLS + off + u * _L, _L)] = sj
                    else:
                        plsc.addupdate(
                            khbuf.at[pl.ds(r * _COLS + off + u * _L, _L)], sj)
                    m = jnp.maximum(1.0 - sj, _EPS)
                    m2 = m * m
                    m4 = m2 * m2
                    bu = b * (m4 * m4 * m2)
                    wbuf[pl.ds(r * _COLS + off + u * _L, _L)] = bu
                    bn.append(bu)
                t = bn
                while len(t) > 1:
                    t = [t[i] + t[i + 1] for i in range(0, len(t), 2)]
                out.append(carry[r] + t[0])
            return tuple(out)

        ssum_v = [_allsum(s) for s in nsm]

    inv_s = [1.0 / s for s in ssum_v]

    @plsc.parallel_loop(0, _COLS, group)
    def _(off):
        for r in range(_RPW):
            for u in range(_MU):
                plsc.addupdate(khbuf.at[pl.ds(r * _COLS + off + u * _L, _L)],
                               wbuf[pl.ds(r * _COLS + off + u * _L, _L)]
                               * inv_s[r])

    pltpu.sync_copy(khbuf, out_hbm.at[pl.ds(base, _WSIZE)])


_run = pl.kernel(
    _sc_body,
    out_type=jax.ShapeDtypeStruct((_ROWS * _COLS,), jnp.float32),
    mesh=plsc.VectorSubcoreMesh(core_axis_name="c", subcore_axis_name="s"),
    scratch_types=[
        pltpu.VMEM((_WSIZE,), jnp.float32),
        pltpu.VMEM((_WSIZE,), jnp.float32),
    ],
)

def _rotl32(x, d):
    return ((x << np.uint32(d)) | (x >> np.uint32(32 - d))).astype(np.uint32)


def _threefry2x32(k0, k1, x0, x1):
    x0 = x0.astype(np.uint32).copy()
    x1 = x1.astype(np.uint32).copy()
    ks = [np.uint32(k0), np.uint32(k1),
          np.uint32(np.uint32(k0) ^ np.uint32(k1) ^ np.uint32(0x1BD11BDA))]
    rots = [[13, 15, 26, 6], [17, 29, 16, 24]]
    with np.errstate(over="ignore"):
        x0 += ks[0]
        x1 += ks[1]
        for i in range(5):
            for r in rots[i % 2]:
                x0 += x1
                x1 = _rotl32(x1, r)
                x1 ^= x0
            x0 += ks[(i + 1) % 3]
            x1 += ks[(i + 2) % 3] + np.uint32(i + 1)
    return x0, x1


_Z_CONST = None


def _gumbel_z():
    global _Z_CONST
    if _Z_CONST is None:
        k0, k1 = _threefry2x32(0, 0, np.zeros(1, np.uint32),
                               np.full(1, 12345, np.uint32))
        counts = np.arange(_ROWS * _COLS, dtype=np.uint64)
        hi = (counts >> np.uint64(32)).astype(np.uint32)
        lo = (counts & np.uint64(0xFFFFFFFF)).astype(np.uint32)
        o0, o1 = _threefry2x32(k0[0], k1[0], hi, lo)
        bits = o0 ^ o1
        f = ((bits >> np.uint32(9)) | np.uint32(0x3F800000)).view(np.float32)
        u = (f - np.float32(1.0)) * np.float32(1.0 - _EPS) + np.float32(_EPS)
        u = np.maximum(np.float32(_EPS), u)
        _Z_CONST = np.log(-np.log(u))
    return _Z_CONST


def kernel(logits):
    w = jnp.reshape(logits, (_ROWS * _COLS,))
    a = (w + jnp.asarray(_gumbel_z())) / _TAU
    out = _run(a)
    return jnp.reshape(out, (_ROWS, _COLS, 1))
